# baseline (speedup 1.0000x reference)
import functools

import jax
import jax.numpy as jnp
from jax import lax
from jax.experimental import pallas as pl
from jax.experimental.pallas import tpu as pltpu
from jax.experimental.pallas import tpu_sc as plsc

N = 10000
E = 160000
DIM = 256
RNUM = 64
NUM_ENT = 10000

NC = 2
NS = 16
NW = NC * NS
LANES = 16

NPAD = 10240
DUMMY = N
EPAD = 163840
CH = 128
GCH = 40
SCH = 80
EB = 640
NB = 1024

_HIGH = lax.Precision.HIGHEST

_sc_mesh = plsc.VectorSubcoreMesh(
    core_axis_name="c", subcore_axis_name="s", num_cores=NC, num_subcores=NS)


def _gather_body(n_chunks, ch, table, idx_hbm, out, idx_v, buf0, buf1,
                 sem0, sem1):
  c = lax.axis_index("c")
  s = lax.axis_index("s")
  w = s * NC + c
  pltpu.sync_copy(idx_hbm.at[w], idx_v)
  rows_per_w = n_chunks * ch
  base = w * rows_per_w

  pltpu.async_copy(table.at[idx_v.at[0]], buf0, sem0)

  def step(g, carry):
    j0 = 2 * g
    j1 = j0 + 1
    pltpu.async_copy(table.at[idx_v.at[j1]], buf1, sem1)
    pltpu.make_async_copy(table.at[pl.ds(0, ch)], buf0, sem0).wait()
    pltpu.sync_copy(buf0, out.at[pl.ds(base + j0 * ch, ch)])

    @pl.when(j1 + 1 < n_chunks)
    def _():
      pltpu.async_copy(table.at[idx_v.at[j1 + 1]], buf0, sem0)

    pltpu.make_async_copy(table.at[pl.ds(0, ch)], buf1, sem1).wait()
    pltpu.sync_copy(buf1, out.at[pl.ds(base + j1 * ch, ch)])
    return carry

  lax.fori_loop(0, n_chunks // 2, step, 0)


def _make_gather(n_chunks, ch, out_rows):
  body = functools.partial(_gather_body, n_chunks, ch)
  return pl.kernel(
      body,
      out_type=jax.ShapeDtypeStruct((out_rows, DIM), jnp.float32),
      mesh=_sc_mesh,
      compiler_params=pltpu.CompilerParams(needs_layout_passes=False),
      scratch_types=[
          pltpu.VMEM((n_chunks, ch), jnp.int32),
          pltpu.VMEM((ch, DIM), jnp.float32),
          pltpu.VMEM((ch, DIM), jnp.float32),
          pltpu.SemaphoreType.DMA,
          pltpu.SemaphoreType.DMA,
      ],
  )


_gather_perm = _make_gather(4, 80, NPAD)


def _scatter_body(sch, msg, idx_hbm, init, out, idx_v, buf0, buf1, acc,
                  sem0, sem1):
  c = lax.axis_index("c")
  s = lax.axis_index("s")
  colbase = c * 128
  ebase = s * (sch * CH)
  pltpu.sync_copy(idx_hbm.at[s], idx_v)

  @pl.when(s == 0)
  def _():
    pltpu.sync_copy(init.at[:, pl.ds(colbase, 128)], acc)

  plsc.subcore_barrier()

  pltpu.async_copy(msg.at[pl.ds(ebase, CH), pl.ds(colbase, 128)], buf0, sem0)

  def step(g, carry):
    j0 = 2 * g
    j1 = j0 + 1
    pltpu.async_copy(
        msg.at[pl.ds(ebase + j1 * CH, CH), pl.ds(colbase, 128)], buf1, sem1)
    pltpu.make_async_copy(
        msg.at[pl.ds(0, CH), pl.ds(0, 128)], buf0, sem0).wait()
    pltpu.sync_copy(buf0, acc.at[idx_v.at[j0]], add=True)

    @pl.when(j1 + 1 < sch)
    def _():
      pltpu.async_copy(
          msg.at[pl.ds(ebase + (j1 + 1) * CH, CH), pl.ds(colbase, 128)],
          buf0, sem0)

    pltpu.make_async_copy(
        msg.at[pl.ds(0, CH), pl.ds(0, 128)], buf1, sem1).wait()
    pltpu.sync_copy(buf1, acc.at[idx_v.at[j1]], add=True)
    return carry

  lax.fori_loop(0, sch // 2, step, 0)
  plsc.subcore_barrier()
  rows = NPAD // NS
  pltpu.sync_copy(
      acc.at[pl.ds(s * rows, rows)],
      out.at[pl.ds(s * rows, rows), pl.ds(colbase, 128)])


def _make_scatter(sch):
  return pl.kernel(
      functools.partial(_scatter_body, sch),
      out_type=jax.ShapeDtypeStruct((NPAD, DIM), jnp.float32),
      mesh=_sc_mesh,
      compiler_params=pltpu.CompilerParams(needs_layout_passes=False),
      scratch_types=[
          pltpu.VMEM((sch, CH), jnp.int32),
          pltpu.VMEM((CH, 128), jnp.float32),
          pltpu.VMEM((CH, 128), jnp.float32),
          pltpu.VMEM_SHARED((NPAD, 128), jnp.float32),
          pltpu.SemaphoreType.DMA,
          pltpu.SemaphoreType.DMA,
      ],
  )


KSLICE = 4
ESLICE = EPAD // KSLICE
_GCHS = ESLICE // (NW * CH)
_SCHS = ESLICE // (NS * CH)
_scatter_slice = _make_scatter(_SCHS)
_gather_slice = _make_gather(_GCHS, CH, ESLICE)


def _mask_body(gamma_hbm, src_hbm, dst_hbm, out, gam_v, src_v, dst_v, idx2_v):
  c = lax.axis_index("c")
  s = lax.axis_index("s")
  w = s * NC + c
  pltpu.sync_copy(gamma_hbm, gam_v)
  pltpu.sync_copy(src_hbm.at[w], src_v)
  pltpu.sync_copy(dst_hbm.at[w], dst_v)

  def step(k, carry):
    j = k // (CH // LANES)
    l = (k % (CH // LANES)) * LANES
    si = src_v[j, pl.ds(l, LANES)]
    di = dst_v[j, pl.ds(l, LANES)]
    gs = plsc.load_gather(gam_v, [si])
    gd = plsc.load_gather(gam_v, [di])
    keep = (gs + gd) / 2.0 > 0.5
    idx2_v[j, pl.ds(l, LANES)] = jnp.where(keep, di, jnp.int32(DUMMY))
    return carry

  lax.fori_loop(0, GCH * (CH // LANES), step, 0)
  pltpu.sync_copy(idx2_v, out.at[w])


_mask_kernel = pl.kernel(
    _mask_body,
    out_type=jax.ShapeDtypeStruct((NW, GCH, CH), jnp.int32),
    mesh=_sc_mesh,
    compiler_params=pltpu.CompilerParams(needs_layout_passes=False),
    scratch_types=[
        pltpu.VMEM((NPAD,), jnp.float32),
        pltpu.VMEM((GCH, CH), jnp.int32),
        pltpu.VMEM((GCH, CH), jnp.int32),
        pltpu.VMEM((GCH, CH), jnp.int32),
    ],
)


def _msg_body(g_ref, t_ref, et_ref, rel_ref, freq_ref, w1_ref, out_ref):
  t = t_ref[...]
  tfeat = jnp.sin(t * freq_ref[...])
  et = et_ref[...]
  oh = (et == lax.broadcasted_iota(jnp.int32, (EB, RNUM), 1)).astype(jnp.float32)
  relpart = jnp.dot(oh, rel_ref[...], precision=_HIGH,
                    preferred_element_type=jnp.float32)
  x = g_ref[...] + relpart + tfeat
  out_ref[...] = jnp.maximum(
      jnp.dot(x, w1_ref[...], preferred_element_type=jnp.float32), 0.0)


def _tc_msg(g, t2, et2, rel_emb, freq2, w1, rows):
  grid = rows // EB
  return pl.pallas_call(
      _msg_body,
      grid=(grid,),
      in_specs=[
          pl.BlockSpec((EB, DIM), lambda i: (i, 0)),
          pl.BlockSpec((EB, 1), lambda i: (i, 0)),
          pl.BlockSpec((EB, 1), lambda i: (i, 0)),
          pl.BlockSpec((RNUM, DIM), lambda i: (0, 0)),
          pl.BlockSpec((1, DIM), lambda i: (0, 0)),
          pl.BlockSpec((DIM, DIM), lambda i: (0, 0)),
      ],
      out_specs=pl.BlockSpec((EB, DIM), lambda i: (i, 0)),
      out_shape=jax.ShapeDtypeStruct((rows, DIM), jnp.float32),
  )(g, t2, et2, rel_emb, freq2, w1)


def _gamma_body(s_ref, w2_ref, mw1_ref, mb1_ref, mw2_ref, mb2_ref, out_ref):
  h = jnp.maximum(jnp.dot(s_ref[...], w2_ref[...],
                          preferred_element_type=jnp.float32), 0.0)
  u = jnp.maximum(jnp.dot(h, mw1_ref[...],
                          preferred_element_type=jnp.float32) + mb1_ref[...], 0.0)
  logit = jnp.dot(u, mw2_ref[...],
                  preferred_element_type=jnp.float32) + mb2_ref[...]
  out_ref[...] = jax.nn.sigmoid(logit)


def _tc_gamma(s, w2, mw1, mb1, mw2, mb2):
  grid = NPAD // NB
  return pl.pallas_call(
      _gamma_body,
      grid=(grid,),
      in_specs=[
          pl.BlockSpec((NB, DIM), lambda i: (i, 0)),
          pl.BlockSpec((DIM, DIM), lambda i: (0, 0)),
          pl.BlockSpec((DIM, DIM), lambda i: (0, 0)),
          pl.BlockSpec((1, DIM), lambda i: (0, 0)),
          pl.BlockSpec((DIM, 1), lambda i: (0, 0)),
          pl.BlockSpec((1, 1), lambda i: (0, 0)),
      ],
      out_specs=pl.BlockSpec((NB, 1), lambda i: (i, 0)),
      out_shape=jax.ShapeDtypeStruct((NPAD, 1), jnp.float32),
  )(s, w2, mw1, mb1, mw2, mb2)


def _hchs_body(s_ref, sc_ref, ne_ref, w2_ref, hc_ref, hs_ref):
  w2 = w2_ref[...]
  hc_ref[...] = jnp.maximum(
      jnp.dot(sc_ref[...], w2, preferred_element_type=jnp.float32), 0.0)
  ss = s_ref[...] - sc_ref[...] + ne_ref[...]
  hs_ref[...] = jnp.maximum(
      jnp.dot(ss, w2, preferred_element_type=jnp.float32), 0.0)


def _tc_hchs(s, sc, ne, w2):
  grid = NPAD // NB
  spec = pl.BlockSpec((NB, DIM), lambda i: (i, 0))
  return pl.pallas_call(
      _hchs_body,
      grid=(grid,),
      in_specs=[spec, spec, spec, pl.BlockSpec((DIM, DIM), lambda i: (0, 0))],
      out_specs=[spec, spec],
      out_shape=[jax.ShapeDtypeStruct((NPAD, DIM), jnp.float32)] * 2,
  )(s, sc, ne, w2)


PM = 400


def _pred_body(hc_ref, hd_ref, pw_ref, pb_ref, out_ref):
  h = (hc_ref[...] + hd_ref[...]).astype(jnp.bfloat16)
  out_ref[...] = jnp.dot(h, pw_ref[...],
                         preferred_element_type=jnp.float32) + pb_ref[...]


def _tc_pred(hc, hd, pw, pb):
  return pl.pallas_call(
      _pred_body,
      grid=(N // PM,),
      in_specs=[
          pl.BlockSpec((PM, DIM), lambda i: (i, 0)),
          pl.BlockSpec((PM, DIM), lambda i: (i, 0)),
          pl.BlockSpec((DIM, NUM_ENT), lambda i: (0, 0)),
          pl.BlockSpec((1, NUM_ENT), lambda i: (0, 0)),
      ],
      out_specs=pl.BlockSpec((PM, NUM_ENT), lambda i: (i, 0)),
      out_shape=jax.ShapeDtypeStruct((N, NUM_ENT), jnp.float32),
  )(hc, hd, pw, pb)


def kernel(edge_index, edge_type, edge_time, node_emb, rel_emb, freq, W1, W2,
           mg_w1, mg_b1, mg_w2, mg_b2, pred_w, pred_b):
  src = edge_index[0].astype(jnp.int32)
  dst = edge_index[1].astype(jnp.int32)
  pad = EPAD - E
  zi = jnp.zeros((pad,), jnp.int32)
  src_pad = jnp.concatenate([src, zi])
  dst_pad = jnp.concatenate([dst, jnp.full((pad,), DUMMY, jnp.int32)])
  src_g = src_pad.reshape(NW, GCH, CH)
  dst_g = dst_pad.reshape(NW, GCH, CH)
  dst_s = dst_pad.reshape(NS, SCH, CH)
  t2 = jnp.concatenate([edge_time, jnp.zeros((pad,), jnp.float32)])[:, None]
  et2 = jnp.concatenate([edge_type.astype(jnp.int32), zi])[:, None]
  ne_pad = jnp.concatenate(
      [node_emb, jnp.zeros((NPAD - N, DIM), jnp.float32)])
  freq2 = freq[None, :]
  mb1 = mg_b1[None, :]
  mb2 = mg_b2.reshape(1, 1)
  perm = jax.random.permutation(jax.random.key(42), N).astype(jnp.int32)
  perm_pad = jnp.concatenate(
      [perm, jnp.zeros((NPAD - N,), jnp.int32)]).reshape(NW, 4, 80)

  msgs = []
  s_acc = ne_pad
  for k in range(KSLICE):
    sl = slice(k * ESLICE, (k + 1) * ESLICE)
    g_k = _gather_slice(node_emb, src_pad[sl].reshape(NW, _GCHS, CH))
    msg_k = _tc_msg(g_k, t2[sl], et2[sl], rel_emb, freq2, W1, ESLICE)
    msgs.append(msg_k)
  for k in range(KSLICE):
    sl = slice(k * ESLICE, (k + 1) * ESLICE)
    s_acc = _scatter_slice(msgs[k], dst_pad[sl].reshape(NS, _SCHS, CH), s_acc)
  s_full = s_acc
  gamma_pad = _tc_gamma(s_full, W2, mg_w1, mb1, mg_w2, mb2)
  idx2 = _mask_kernel(gamma_pad.reshape(NPAD), src_g, dst_g)
  idx2_flat = idx2.reshape(EPAD)
  sc_acc = ne_pad
  for k in range(KSLICE):
    sl = slice(k * ESLICE, (k + 1) * ESLICE)
    sc_acc = _scatter_slice(
        msgs[k], idx2_flat[sl].reshape(NS, _SCHS, CH), sc_acc)
  s_c = sc_acc
  hc_pad, hs_pad = _tc_hchs(s_full, s_c, ne_pad, W2)
  hs_do = _gather_perm(hs_pad, perm_pad)
  scores = _tc_pred(hc_pad[:N], hs_do[:N], pred_w.astype(jnp.bfloat16),
                    pred_b.reshape(1, NUM_ENT))
  return (scores, gamma_pad[:N], hc_pad[:N], hs_pad[:N])


# speedup vs baseline: 1.0483x; 1.0483x over previous
import functools

import jax
import jax.numpy as jnp
from jax import lax
from jax.experimental import pallas as pl
from jax.experimental.pallas import tpu as pltpu
from jax.experimental.pallas import tpu_sc as plsc

N = 10000
E = 160000
DIM = 256
RNUM = 64
NUM_ENT = 10000

NC = 2
NS = 16
NW = NC * NS
LANES = 16

NPAD = 10240
DUMMY = N
EPAD = 163840
CH = 128
GCH = 40
SCH = 80
EB = 640
NB = 1024

_HIGH = lax.Precision.HIGHEST

_sc_mesh = plsc.VectorSubcoreMesh(
    core_axis_name="c", subcore_axis_name="s", num_cores=NC, num_subcores=NS)


def _gather_body(n_chunks, ch, table, idx_hbm, out, idx_v, buf0, buf1,
                 sem0, sem1):
  c = lax.axis_index("c")
  s = lax.axis_index("s")
  w = s * NC + c
  pltpu.sync_copy(idx_hbm.at[w], idx_v)
  rows_per_w = n_chunks * ch
  base = w * rows_per_w

  pltpu.async_copy(table.at[idx_v.at[0]], buf0, sem0)

  def step(g, carry):
    j0 = 2 * g
    j1 = j0 + 1
    pltpu.async_copy(table.at[idx_v.at[j1]], buf1, sem1)
    pltpu.make_async_copy(table.at[pl.ds(0, ch)], buf0, sem0).wait()
    pltpu.sync_copy(buf0, out.at[pl.ds(base + j0 * ch, ch)])

    @pl.when(j1 + 1 < n_chunks)
    def _():
      pltpu.async_copy(table.at[idx_v.at[j1 + 1]], buf0, sem0)

    pltpu.make_async_copy(table.at[pl.ds(0, ch)], buf1, sem1).wait()
    pltpu.sync_copy(buf1, out.at[pl.ds(base + j1 * ch, ch)])
    return carry

  lax.fori_loop(0, n_chunks // 2, step, 0)


def _make_gather(n_chunks, ch, out_rows):
  body = functools.partial(_gather_body, n_chunks, ch)
  return pl.kernel(
      body,
      out_type=jax.ShapeDtypeStruct((out_rows, DIM), jnp.float32),
      mesh=_sc_mesh,
      compiler_params=pltpu.CompilerParams(needs_layout_passes=False),
      scratch_types=[
          pltpu.VMEM((n_chunks, ch), jnp.int32),
          pltpu.VMEM((ch, DIM), jnp.float32),
          pltpu.VMEM((ch, DIM), jnp.float32),
          pltpu.SemaphoreType.DMA,
          pltpu.SemaphoreType.DMA,
      ],
  )


_gather_perm = _make_gather(4, 80, NPAD)


def _scatter_body(sch, msg, idx_hbm, init, out, idx_v, buf0, buf1, acc,
                  sem0, sem1):
  c = lax.axis_index("c")
  s = lax.axis_index("s")
  colbase = c * 128
  ebase = s * (sch * CH)
  pltpu.sync_copy(idx_hbm.at[s], idx_v)

  @pl.when(s == 0)
  def _():
    pltpu.sync_copy(init.at[:, pl.ds(colbase, 128)], acc)

  plsc.subcore_barrier()

  pltpu.async_copy(msg.at[pl.ds(ebase, CH), pl.ds(colbase, 128)], buf0, sem0)

  def step(g, carry):
    j0 = 2 * g
    j1 = j0 + 1
    pltpu.async_copy(
        msg.at[pl.ds(ebase + j1 * CH, CH), pl.ds(colbase, 128)], buf1, sem1)
    pltpu.make_async_copy(
        msg.at[pl.ds(0, CH), pl.ds(0, 128)], buf0, sem0).wait()
    pltpu.sync_copy(buf0, acc.at[idx_v.at[j0]], add=True)

    @pl.when(j1 + 1 < sch)
    def _():
      pltpu.async_copy(
          msg.at[pl.ds(ebase + (j1 + 1) * CH, CH), pl.ds(colbase, 128)],
          buf0, sem0)

    pltpu.make_async_copy(
        msg.at[pl.ds(0, CH), pl.ds(0, 128)], buf1, sem1).wait()
    pltpu.sync_copy(buf1, acc.at[idx_v.at[j1]], add=True)
    return carry

  lax.fori_loop(0, sch // 2, step, 0)
  plsc.subcore_barrier()
  rows = NPAD // NS
  pltpu.sync_copy(
      acc.at[pl.ds(s * rows, rows)],
      out.at[pl.ds(s * rows, rows), pl.ds(colbase, 128)])


def _make_scatter(sch):
  return pl.kernel(
      functools.partial(_scatter_body, sch),
      out_type=jax.ShapeDtypeStruct((NPAD, DIM), jnp.float32),
      mesh=_sc_mesh,
      compiler_params=pltpu.CompilerParams(needs_layout_passes=False),
      scratch_types=[
          pltpu.VMEM((sch, CH), jnp.int32),
          pltpu.VMEM((CH, 128), jnp.float32),
          pltpu.VMEM((CH, 128), jnp.float32),
          pltpu.VMEM_SHARED((NPAD, 128), jnp.float32),
          pltpu.SemaphoreType.DMA,
          pltpu.SemaphoreType.DMA,
      ],
  )


def _scatter2_body(sch, msg0, msg1, idx_hbm, init, out, idx_v, buf0, buf1,
                   acc, sem0, sem1):
  c = lax.axis_index("c")
  s = lax.axis_index("s")
  colbase = c * 128
  ebase = s * (sch * CH)
  pltpu.sync_copy(idx_hbm.at[s], idx_v)

  @pl.when(s == 0)
  def _():
    pltpu.sync_copy(init.at[:, pl.ds(colbase, 128)], acc)

  plsc.subcore_barrier()

  def half(msg, joff):
    pltpu.async_copy(msg.at[pl.ds(ebase, CH), pl.ds(colbase, 128)], buf0, sem0)

    def step(g, carry):
      j0 = 2 * g
      j1 = j0 + 1
      pltpu.async_copy(
          msg.at[pl.ds(ebase + j1 * CH, CH), pl.ds(colbase, 128)], buf1, sem1)
      pltpu.make_async_copy(
          msg.at[pl.ds(0, CH), pl.ds(0, 128)], buf0, sem0).wait()
      pltpu.sync_copy(buf0, acc.at[idx_v.at[joff + j0]], add=True)

      @pl.when(j1 + 1 < sch)
      def _():
        pltpu.async_copy(
            msg.at[pl.ds(ebase + (j1 + 1) * CH, CH), pl.ds(colbase, 128)],
            buf0, sem0)

      pltpu.make_async_copy(
          msg.at[pl.ds(0, CH), pl.ds(0, 128)], buf1, sem1).wait()
      pltpu.sync_copy(buf1, acc.at[idx_v.at[joff + j1]], add=True)
      return carry

    lax.fori_loop(0, sch // 2, step, 0)

  half(msg0, 0)
  half(msg1, sch)
  plsc.subcore_barrier()
  rows = NPAD // NS
  pltpu.sync_copy(
      acc.at[pl.ds(s * rows, rows)],
      out.at[pl.ds(s * rows, rows), pl.ds(colbase, 128)])


def _make_scatter2(sch):
  return pl.kernel(
      functools.partial(_scatter2_body, sch),
      out_type=jax.ShapeDtypeStruct((NPAD, DIM), jnp.float32),
      mesh=_sc_mesh,
      compiler_params=pltpu.CompilerParams(needs_layout_passes=False),
      scratch_types=[
          pltpu.VMEM((2 * sch, CH), jnp.int32),
          pltpu.VMEM((CH, 128), jnp.float32),
          pltpu.VMEM((CH, 128), jnp.float32),
          pltpu.VMEM_SHARED((NPAD, 128), jnp.float32),
          pltpu.SemaphoreType.DMA,
          pltpu.SemaphoreType.DMA,
      ],
  )


KSLICE = 2
ESLICE = EPAD // KSLICE
_GCHS = ESLICE // (NW * CH)
_SCHS = ESLICE // (NS * CH)
_scatter_slice = _make_scatter(_SCHS)
_scatter_masked = _make_scatter2(_SCHS)
_gather_slice = _make_gather(_GCHS, CH, ESLICE)


def _mask_body(gamma_hbm, src_hbm, dst_hbm, out, gam_v, src_v, dst_v, idx2_v):
  c = lax.axis_index("c")
  s = lax.axis_index("s")
  w = s * NC + c
  pltpu.sync_copy(gamma_hbm, gam_v)
  pltpu.sync_copy(src_hbm.at[w], src_v)
  pltpu.sync_copy(dst_hbm.at[w], dst_v)

  def step(k, carry):
    j = k // (CH // LANES)
    l = (k % (CH // LANES)) * LANES
    si = src_v[j, pl.ds(l, LANES)]
    di = dst_v[j, pl.ds(l, LANES)]
    gs = plsc.load_gather(gam_v, [si])
    gd = plsc.load_gather(gam_v, [di])
    keep = (gs + gd) / 2.0 > 0.5
    idx2_v[j, pl.ds(l, LANES)] = jnp.where(keep, di, jnp.int32(DUMMY))
    return carry

  lax.fori_loop(0, GCH * (CH // LANES), step, 0)
  pltpu.sync_copy(idx2_v, out.at[w])


_mask_kernel = pl.kernel(
    _mask_body,
    out_type=jax.ShapeDtypeStruct((NW, GCH, CH), jnp.int32),
    mesh=_sc_mesh,
    compiler_params=pltpu.CompilerParams(needs_layout_passes=False),
    scratch_types=[
        pltpu.VMEM((NPAD,), jnp.float32),
        pltpu.VMEM((GCH, CH), jnp.int32),
        pltpu.VMEM((GCH, CH), jnp.int32),
        pltpu.VMEM((GCH, CH), jnp.int32),
    ],
)


def _msg_body(g_ref, t_ref, et_ref, rel_ref, freq_ref, w1_ref, out_ref):
  t = t_ref[...]
  tfeat = jnp.sin(t * freq_ref[...])
  et = et_ref[...]
  oh = (et == lax.broadcasted_iota(jnp.int32, (EB, RNUM), 1)).astype(jnp.float32)
  relpart = jnp.dot(oh, rel_ref[...], precision=_HIGH,
                    preferred_element_type=jnp.float32)
  x = g_ref[...] + relpart + tfeat
  out_ref[...] = jnp.maximum(
      jnp.dot(x, w1_ref[...], preferred_element_type=jnp.float32), 0.0)


def _tc_msg(g, t2, et2, rel_emb, freq2, w1, rows):
  grid = rows // EB
  return pl.pallas_call(
      _msg_body,
      grid=(grid,),
      in_specs=[
          pl.BlockSpec((EB, DIM), lambda i: (i, 0)),
          pl.BlockSpec((EB, 1), lambda i: (i, 0)),
          pl.BlockSpec((EB, 1), lambda i: (i, 0)),
          pl.BlockSpec((RNUM, DIM), lambda i: (0, 0)),
          pl.BlockSpec((1, DIM), lambda i: (0, 0)),
          pl.BlockSpec((DIM, DIM), lambda i: (0, 0)),
      ],
      out_specs=pl.BlockSpec((EB, DIM), lambda i: (i, 0)),
      out_shape=jax.ShapeDtypeStruct((rows, DIM), jnp.float32),
  )(g, t2, et2, rel_emb, freq2, w1)


def _gamma_body(s_ref, w2_ref, mw1_ref, mb1_ref, mw2_ref, mb2_ref, out_ref):
  h = jnp.maximum(jnp.dot(s_ref[...], w2_ref[...],
                          preferred_element_type=jnp.float32), 0.0)
  u = jnp.maximum(jnp.dot(h, mw1_ref[...],
                          preferred_element_type=jnp.float32) + mb1_ref[...], 0.0)
  logit = jnp.dot(u, mw2_ref[...],
                  preferred_element_type=jnp.float32) + mb2_ref[...]
  out_ref[...] = jax.nn.sigmoid(logit)


def _tc_gamma(s, w2, mw1, mb1, mw2, mb2):
  grid = NPAD // NB
  return pl.pallas_call(
      _gamma_body,
      grid=(grid,),
      in_specs=[
          pl.BlockSpec((NB, DIM), lambda i: (i, 0)),
          pl.BlockSpec((DIM, DIM), lambda i: (0, 0)),
          pl.BlockSpec((DIM, DIM), lambda i: (0, 0)),
          pl.BlockSpec((1, DIM), lambda i: (0, 0)),
          pl.BlockSpec((DIM, 1), lambda i: (0, 0)),
          pl.BlockSpec((1, 1), lambda i: (0, 0)),
      ],
      out_specs=pl.BlockSpec((NB, 1), lambda i: (i, 0)),
      out_shape=jax.ShapeDtypeStruct((NPAD, 1), jnp.float32),
  )(s, w2, mw1, mb1, mw2, mb2)


def _hchs_body(s_ref, sc_ref, ne_ref, w2_ref, hc_ref, hs_ref):
  w2 = w2_ref[...]
  hc_ref[...] = jnp.maximum(
      jnp.dot(sc_ref[...], w2, preferred_element_type=jnp.float32), 0.0)
  ss = s_ref[...] - sc_ref[...] + ne_ref[...]
  hs_ref[...] = jnp.maximum(
      jnp.dot(ss, w2, preferred_element_type=jnp.float32), 0.0)


def _tc_hchs(s, sc, ne, w2):
  grid = NPAD // NB
  spec = pl.BlockSpec((NB, DIM), lambda i: (i, 0))
  return pl.pallas_call(
      _hchs_body,
      grid=(grid,),
      in_specs=[spec, spec, spec, pl.BlockSpec((DIM, DIM), lambda i: (0, 0))],
      out_specs=[spec, spec],
      out_shape=[jax.ShapeDtypeStruct((NPAD, DIM), jnp.float32)] * 2,
  )(s, sc, ne, w2)


PM = 400


def _pred_body(hc_ref, hd_ref, pw_ref, pb_ref, out_ref):
  h = (hc_ref[...] + hd_ref[...]).astype(jnp.bfloat16)
  out_ref[...] = jnp.dot(h, pw_ref[...],
                         preferred_element_type=jnp.float32) + pb_ref[...]


def _tc_pred(hc, hd, pw, pb):
  return pl.pallas_call(
      _pred_body,
      grid=(N // PM,),
      in_specs=[
          pl.BlockSpec((PM, DIM), lambda i: (i, 0)),
          pl.BlockSpec((PM, DIM), lambda i: (i, 0)),
          pl.BlockSpec((DIM, NUM_ENT), lambda i: (0, 0)),
          pl.BlockSpec((1, NUM_ENT), lambda i: (0, 0)),
      ],
      out_specs=pl.BlockSpec((PM, NUM_ENT), lambda i: (i, 0)),
      out_shape=jax.ShapeDtypeStruct((N, NUM_ENT), jnp.float32),
  )(hc, hd, pw, pb)


def kernel(edge_index, edge_type, edge_time, node_emb, rel_emb, freq, W1, W2,
           mg_w1, mg_b1, mg_w2, mg_b2, pred_w, pred_b):
  src = edge_index[0].astype(jnp.int32)
  dst = edge_index[1].astype(jnp.int32)
  pad = EPAD - E
  zi = jnp.zeros((pad,), jnp.int32)
  src_pad = jnp.concatenate([src, zi])
  dst_pad = jnp.concatenate([dst, jnp.full((pad,), DUMMY, jnp.int32)])
  src_g = src_pad.reshape(NW, GCH, CH)
  dst_g = dst_pad.reshape(NW, GCH, CH)
  dst_s = dst_pad.reshape(NS, SCH, CH)
  t2 = jnp.concatenate([edge_time, jnp.zeros((pad,), jnp.float32)])[:, None]
  et2 = jnp.concatenate([edge_type.astype(jnp.int32), zi])[:, None]
  ne_pad = jnp.concatenate(
      [node_emb, jnp.zeros((NPAD - N, DIM), jnp.float32)])
  freq2 = freq[None, :]
  mb1 = mg_b1[None, :]
  mb2 = mg_b2.reshape(1, 1)
  perm = jax.random.permutation(jax.random.key(42), N).astype(jnp.int32)
  perm_pad = jnp.concatenate(
      [perm, jnp.zeros((NPAD - N,), jnp.int32)]).reshape(NW, 4, 80)

  msgs = []
  s_acc = ne_pad
  for k in range(KSLICE):
    sl = slice(k * ESLICE, (k + 1) * ESLICE)
    g_k = _gather_slice(node_emb, src_pad[sl].reshape(NW, _GCHS, CH))
    msg_k = _tc_msg(g_k, t2[sl], et2[sl], rel_emb, freq2, W1, ESLICE)
    msgs.append(msg_k)
  for k in range(KSLICE):
    sl = slice(k * ESLICE, (k + 1) * ESLICE)
    s_acc = _scatter_slice(msgs[k], dst_pad[sl].reshape(NS, _SCHS, CH), s_acc)
  s_full = s_acc
  gamma_pad = _tc_gamma(s_full, W2, mg_w1, mb1, mg_w2, mb2)
  idx2 = _mask_kernel(gamma_pad.reshape(NPAD), src_g, dst_g)
  idx2_flat = idx2.reshape(EPAD)
  idx2_both = jnp.concatenate(
      [idx2_flat[:ESLICE].reshape(NS, _SCHS, CH),
       idx2_flat[ESLICE:].reshape(NS, _SCHS, CH)], axis=1)
  s_c = _scatter_masked(msgs[0], msgs[1], idx2_both, ne_pad)
  hc_pad, hs_pad = _tc_hchs(s_full, s_c, ne_pad, W2)
  hs_do = _gather_perm(hs_pad, perm_pad)
  scores = _tc_pred(hc_pad[:N], hs_do[:N], pred_w.astype(jnp.bfloat16),
                    pred_b.reshape(1, NUM_ENT))
  return (scores, gamma_pad[:N], hc_pad[:N], hs_pad[:N])


# speedup vs baseline: 1.1066x; 1.0556x over previous
import functools

import jax
import jax.numpy as jnp
from jax import lax
from jax.experimental import pallas as pl
from jax.experimental.pallas import tpu as pltpu
from jax.experimental.pallas import tpu_sc as plsc

N = 10000
E = 160000
DIM = 256
RNUM = 64
NUM_ENT = 10000

NC = 2
NS = 16
NW = NC * NS
LANES = 16

NPAD = 10240
DUMMY = N
EPAD = 163840
CH = 128
GCH = 40
SCH = 80
EB = 1280
NB = 1024

_HIGH = lax.Precision.HIGHEST

_sc_mesh = plsc.VectorSubcoreMesh(
    core_axis_name="c", subcore_axis_name="s", num_cores=NC, num_subcores=NS)


def _gather_body(n_chunks, ch, table, idx_hbm, out, idx_v, buf0, buf1,
                 sem0, sem1):
  c = lax.axis_index("c")
  s = lax.axis_index("s")
  w = s * NC + c
  pltpu.sync_copy(idx_hbm.at[w], idx_v)
  rows_per_w = n_chunks * ch
  base = w * rows_per_w

  pltpu.async_copy(table.at[idx_v.at[0]], buf0, sem0)

  def step(g, carry):
    j0 = 2 * g
    j1 = j0 + 1
    pltpu.async_copy(table.at[idx_v.at[j1]], buf1, sem1)
    pltpu.make_async_copy(table.at[pl.ds(0, ch)], buf0, sem0).wait()
    pltpu.sync_copy(buf0, out.at[pl.ds(base + j0 * ch, ch)])

    @pl.when(j1 + 1 < n_chunks)
    def _():
      pltpu.async_copy(table.at[idx_v.at[j1 + 1]], buf0, sem0)

    pltpu.make_async_copy(table.at[pl.ds(0, ch)], buf1, sem1).wait()
    pltpu.sync_copy(buf1, out.at[pl.ds(base + j1 * ch, ch)])
    return carry

  lax.fori_loop(0, n_chunks // 2, step, 0)


def _make_gather(n_chunks, ch, out_rows):
  body = functools.partial(_gather_body, n_chunks, ch)
  return pl.kernel(
      body,
      out_type=jax.ShapeDtypeStruct((out_rows, DIM), jnp.float32),
      mesh=_sc_mesh,
      compiler_params=pltpu.CompilerParams(needs_layout_passes=False),
      scratch_types=[
          pltpu.VMEM((n_chunks, ch), jnp.int32),
          pltpu.VMEM((ch, DIM), jnp.float32),
          pltpu.VMEM((ch, DIM), jnp.float32),
          pltpu.SemaphoreType.DMA,
          pltpu.SemaphoreType.DMA,
      ],
  )


_gather_perm = _make_gather(4, 80, NPAD)


def _scatter_body(sch, msg, idx_hbm, init, out, idx_v, buf0, buf1, acc,
                  sem0, sem1):
  c = lax.axis_index("c")
  s = lax.axis_index("s")
  colbase = c * 128
  ebase = s * (sch * CH)
  pltpu.sync_copy(idx_hbm.at[s], idx_v)

  @pl.when(s == 0)
  def _():
    pltpu.sync_copy(init.at[:, pl.ds(colbase, 128)], acc)

  plsc.subcore_barrier()

  pltpu.async_copy(msg.at[pl.ds(ebase, CH), pl.ds(colbase, 128)], buf0, sem0)

  def step(g, carry):
    j0 = 2 * g
    j1 = j0 + 1
    pltpu.async_copy(
        msg.at[pl.ds(ebase + j1 * CH, CH), pl.ds(colbase, 128)], buf1, sem1)
    pltpu.make_async_copy(
        msg.at[pl.ds(0, CH), pl.ds(0, 128)], buf0, sem0).wait()
    pltpu.sync_copy(buf0, acc.at[idx_v.at[j0]], add=True)

    @pl.when(j1 + 1 < sch)
    def _():
      pltpu.async_copy(
          msg.at[pl.ds(ebase + (j1 + 1) * CH, CH), pl.ds(colbase, 128)],
          buf0, sem0)

    pltpu.make_async_copy(
        msg.at[pl.ds(0, CH), pl.ds(0, 128)], buf1, sem1).wait()
    pltpu.sync_copy(buf1, acc.at[idx_v.at[j1]], add=True)
    return carry

  lax.fori_loop(0, sch // 2, step, 0)
  plsc.subcore_barrier()
  rows = NPAD // NS
  pltpu.sync_copy(
      acc.at[pl.ds(s * rows, rows)],
      out.at[pl.ds(s * rows, rows), pl.ds(colbase, 128)])


def _make_scatter(sch):
  return pl.kernel(
      functools.partial(_scatter_body, sch),
      out_type=jax.ShapeDtypeStruct((NPAD, DIM), jnp.float32),
      mesh=_sc_mesh,
      compiler_params=pltpu.CompilerParams(needs_layout_passes=False),
      scratch_types=[
          pltpu.VMEM((sch, CH), jnp.int32),
          pltpu.VMEM((CH, 128), jnp.float32),
          pltpu.VMEM((CH, 128), jnp.float32),
          pltpu.VMEM_SHARED((NPAD, 128), jnp.float32),
          pltpu.SemaphoreType.DMA,
          pltpu.SemaphoreType.DMA,
      ],
  )


def _scatter2_body(sch, msg0, msg1, idx_hbm, init, out, idx_v, buf0, buf1,
                   acc, sem0, sem1):
  c = lax.axis_index("c")
  s = lax.axis_index("s")
  colbase = c * 128
  ebase = s * (sch * CH)
  pltpu.sync_copy(idx_hbm.at[s], idx_v)

  @pl.when(s == 0)
  def _():
    pltpu.sync_copy(init.at[:, pl.ds(colbase, 128)], acc)

  plsc.subcore_barrier()

  def half(msg, joff):
    pltpu.async_copy(msg.at[pl.ds(ebase, CH), pl.ds(colbase, 128)], buf0, sem0)

    def step(g, carry):
      j0 = 2 * g
      j1 = j0 + 1
      pltpu.async_copy(
          msg.at[pl.ds(ebase + j1 * CH, CH), pl.ds(colbase, 128)], buf1, sem1)
      pltpu.make_async_copy(
          msg.at[pl.ds(0, CH), pl.ds(0, 128)], buf0, sem0).wait()
      pltpu.sync_copy(buf0, acc.at[idx_v.at[joff + j0]], add=True)

      @pl.when(j1 + 1 < sch)
      def _():
        pltpu.async_copy(
            msg.at[pl.ds(ebase + (j1 + 1) * CH, CH), pl.ds(colbase, 128)],
            buf0, sem0)

      pltpu.make_async_copy(
          msg.at[pl.ds(0, CH), pl.ds(0, 128)], buf1, sem1).wait()
      pltpu.sync_copy(buf1, acc.at[idx_v.at[joff + j1]], add=True)
      return carry

    lax.fori_loop(0, sch // 2, step, 0)

  half(msg0, 0)
  half(msg1, sch)
  plsc.subcore_barrier()
  rows = NPAD // NS
  pltpu.sync_copy(
      acc.at[pl.ds(s * rows, rows)],
      out.at[pl.ds(s * rows, rows), pl.ds(colbase, 128)])


def _make_scatter2(sch):
  return pl.kernel(
      functools.partial(_scatter2_body, sch),
      out_type=jax.ShapeDtypeStruct((NPAD, DIM), jnp.float32),
      mesh=_sc_mesh,
      compiler_params=pltpu.CompilerParams(needs_layout_passes=False),
      scratch_types=[
          pltpu.VMEM((2 * sch, CH), jnp.int32),
          pltpu.VMEM((CH, 128), jnp.float32),
          pltpu.VMEM((CH, 128), jnp.float32),
          pltpu.VMEM_SHARED((NPAD, 128), jnp.float32),
          pltpu.SemaphoreType.DMA,
          pltpu.SemaphoreType.DMA,
      ],
  )


KSLICE = 2
ESLICE = EPAD // KSLICE
_GCHS = ESLICE // (NW * CH)
_SCHS = ESLICE // (NS * CH)
_scatter_slice = _make_scatter(_SCHS)
_scatter_masked = _make_scatter2(_SCHS)
_gather_slice = _make_gather(_GCHS, CH, ESLICE)


def _mask_body(gamma_hbm, src_hbm, dst_hbm, out, gam_v, src_v, dst_v, idx2_v):
  c = lax.axis_index("c")
  s = lax.axis_index("s")
  w = s * NC + c
  pltpu.sync_copy(gamma_hbm, gam_v)
  pltpu.sync_copy(src_hbm.at[w], src_v)
  pltpu.sync_copy(dst_hbm.at[w], dst_v)

  def step(k, carry):
    j = k // (CH // LANES)
    l = (k % (CH // LANES)) * LANES
    si = src_v[j, pl.ds(l, LANES)]
    di = dst_v[j, pl.ds(l, LANES)]
    gs = plsc.load_gather(gam_v, [si])
    gd = plsc.load_gather(gam_v, [di])
    keep = (gs + gd) / 2.0 > 0.5
    idx2_v[j, pl.ds(l, LANES)] = jnp.where(keep, di, jnp.int32(DUMMY))
    return carry

  lax.fori_loop(0, GCH * (CH // LANES), step, 0)
  pltpu.sync_copy(idx2_v, out.at[w])


_mask_kernel = pl.kernel(
    _mask_body,
    out_type=jax.ShapeDtypeStruct((NW, GCH, CH), jnp.int32),
    mesh=_sc_mesh,
    compiler_params=pltpu.CompilerParams(needs_layout_passes=False),
    scratch_types=[
        pltpu.VMEM((NPAD,), jnp.float32),
        pltpu.VMEM((GCH, CH), jnp.int32),
        pltpu.VMEM((GCH, CH), jnp.int32),
        pltpu.VMEM((GCH, CH), jnp.int32),
    ],
)


def _msg_body(g_ref, t_ref, et_ref, rel_ref, freq_ref, w1_ref, out_ref):
  t = t_ref[...]
  tfeat = jnp.sin(t * freq_ref[...])
  et = et_ref[...]
  oh = (et == lax.broadcasted_iota(jnp.int32, (EB, RNUM), 1)).astype(jnp.float32)
  relpart = jnp.dot(oh, rel_ref[...], precision=_HIGH,
                    preferred_element_type=jnp.float32)
  x = g_ref[...] + relpart + tfeat
  out_ref[...] = jnp.maximum(
      jnp.dot(x, w1_ref[...], preferred_element_type=jnp.float32), 0.0)


def _tc_msg(g, t2, et2, rel_emb, freq2, w1, rows):
  grid = rows // EB
  return pl.pallas_call(
      _msg_body,
      grid=(grid,),
      in_specs=[
          pl.BlockSpec((EB, DIM), lambda i: (i, 0)),
          pl.BlockSpec((EB, 1), lambda i: (i, 0)),
          pl.BlockSpec((EB, 1), lambda i: (i, 0)),
          pl.BlockSpec((RNUM, DIM), lambda i: (0, 0)),
          pl.BlockSpec((1, DIM), lambda i: (0, 0)),
          pl.BlockSpec((DIM, DIM), lambda i: (0, 0)),
      ],
      out_specs=pl.BlockSpec((EB, DIM), lambda i: (i, 0)),
      out_shape=jax.ShapeDtypeStruct((rows, DIM), jnp.float32),
  )(g, t2, et2, rel_emb, freq2, w1)


def _gamma_body(s_ref, w2_ref, mw1_ref, mb1_ref, mw2_ref, mb2_ref, out_ref):
  h = jnp.maximum(jnp.dot(s_ref[...], w2_ref[...],
                          preferred_element_type=jnp.float32), 0.0)
  u = jnp.maximum(jnp.dot(h, mw1_ref[...],
                          preferred_element_type=jnp.float32) + mb1_ref[...], 0.0)
  logit = jnp.dot(u, mw2_ref[...],
                  preferred_element_type=jnp.float32) + mb2_ref[...]
  out_ref[...] = jax.nn.sigmoid(logit)


def _tc_gamma(s, w2, mw1, mb1, mw2, mb2):
  grid = NPAD // NB
  return pl.pallas_call(
      _gamma_body,
      grid=(grid,),
      in_specs=[
          pl.BlockSpec((NB, DIM), lambda i: (i, 0)),
          pl.BlockSpec((DIM, DIM), lambda i: (0, 0)),
          pl.BlockSpec((DIM, DIM), lambda i: (0, 0)),
          pl.BlockSpec((1, DIM), lambda i: (0, 0)),
          pl.BlockSpec((DIM, 1), lambda i: (0, 0)),
          pl.BlockSpec((1, 1), lambda i: (0, 0)),
      ],
      out_specs=pl.BlockSpec((NB, 1), lambda i: (i, 0)),
      out_shape=jax.ShapeDtypeStruct((NPAD, 1), jnp.float32),
  )(s, w2, mw1, mb1, mw2, mb2)


def _hchs_body(s_ref, sc_ref, ne_ref, w2_ref, hc_ref, hs_ref):
  w2 = w2_ref[...]
  hc_ref[...] = jnp.maximum(
      jnp.dot(sc_ref[...], w2, preferred_element_type=jnp.float32), 0.0)
  ss = s_ref[...] - sc_ref[...] + ne_ref[...]
  hs_ref[...] = jnp.maximum(
      jnp.dot(ss, w2, preferred_element_type=jnp.float32), 0.0)


def _tc_hchs(s, sc, ne, w2):
  grid = NPAD // NB
  spec = pl.BlockSpec((NB, DIM), lambda i: (i, 0))
  return pl.pallas_call(
      _hchs_body,
      grid=(grid,),
      in_specs=[spec, spec, spec, pl.BlockSpec((DIM, DIM), lambda i: (0, 0))],
      out_specs=[spec, spec],
      out_shape=[jax.ShapeDtypeStruct((NPAD, DIM), jnp.float32)] * 2,
  )(s, sc, ne, w2)


PM = 400


def _pred_body(hc_ref, hd_ref, pw_ref, pb_ref, out_ref):
  h = (hc_ref[...] + hd_ref[...]).astype(jnp.bfloat16)
  out_ref[...] = jnp.dot(h, pw_ref[...],
                         preferred_element_type=jnp.float32) + pb_ref[...]


def _tc_pred(hc, hd, pw, pb):
  return pl.pallas_call(
      _pred_body,
      grid=(N // PM,),
      in_specs=[
          pl.BlockSpec((PM, DIM), lambda i: (i, 0)),
          pl.BlockSpec((PM, DIM), lambda i: (i, 0)),
          pl.BlockSpec((DIM, NUM_ENT), lambda i: (0, 0)),
          pl.BlockSpec((1, NUM_ENT), lambda i: (0, 0)),
      ],
      out_specs=pl.BlockSpec((PM, NUM_ENT), lambda i: (i, 0)),
      out_shape=jax.ShapeDtypeStruct((N, NUM_ENT), jnp.float32),
  )(hc, hd, pw, pb)


def kernel(edge_index, edge_type, edge_time, node_emb, rel_emb, freq, W1, W2,
           mg_w1, mg_b1, mg_w2, mg_b2, pred_w, pred_b):
  src = edge_index[0].astype(jnp.int32)
  dst = edge_index[1].astype(jnp.int32)
  pad = EPAD - E
  zi = jnp.zeros((pad,), jnp.int32)
  src_pad = jnp.concatenate([src, zi])
  dst_pad = jnp.concatenate([dst, jnp.full((pad,), DUMMY, jnp.int32)])
  src_g = src_pad.reshape(NW, GCH, CH)
  dst_g = dst_pad.reshape(NW, GCH, CH)
  dst_s = dst_pad.reshape(NS, SCH, CH)
  t2 = jnp.concatenate([edge_time, jnp.zeros((pad,), jnp.float32)])[:, None]
  et2 = jnp.concatenate([edge_type.astype(jnp.int32), zi])[:, None]
  ne_pad = jnp.concatenate(
      [node_emb, jnp.zeros((NPAD - N, DIM), jnp.float32)])
  freq2 = freq[None, :]
  mb1 = mg_b1[None, :]
  mb2 = mg_b2.reshape(1, 1)
  perm = jax.random.permutation(jax.random.key(42), N).astype(jnp.int32)
  perm_pad = jnp.concatenate(
      [perm, jnp.zeros((NPAD - N,), jnp.int32)]).reshape(NW, 4, 80)

  msgs = []
  s_acc = ne_pad
  for k in range(KSLICE):
    sl = slice(k * ESLICE, (k + 1) * ESLICE)
    g_k = _gather_slice(node_emb, src_pad[sl].reshape(NW, _GCHS, CH))
    msg_k = _tc_msg(g_k, t2[sl], et2[sl], rel_emb, freq2, W1, ESLICE)
    msgs.append(msg_k)
  for k in range(KSLICE):
    sl = slice(k * ESLICE, (k + 1) * ESLICE)
    s_acc = _scatter_slice(msgs[k], dst_pad[sl].reshape(NS, _SCHS, CH), s_acc)
  s_full = s_acc
  gamma_pad = _tc_gamma(s_full, W2, mg_w1, mb1, mg_w2, mb2)
  idx2 = _mask_kernel(gamma_pad.reshape(NPAD), src_g, dst_g)
  idx2_flat = idx2.reshape(EPAD)
  idx2_both = jnp.concatenate(
      [idx2_flat[:ESLICE].reshape(NS, _SCHS, CH),
       idx2_flat[ESLICE:].reshape(NS, _SCHS, CH)], axis=1)
  s_c = _scatter_masked(msgs[0], msgs[1], idx2_both, ne_pad)
  hc_pad, hs_pad = _tc_hchs(s_full, s_c, ne_pad, W2)
  hs_do = _gather_perm(hs_pad, perm_pad)
  scores = _tc_pred(hc_pad[:N], hs_do[:N], pred_w.astype(jnp.bfloat16),
                    pred_b.reshape(1, NUM_ENT))
  return (scores, gamma_pad[:N], hc_pad[:N], hs_pad[:N])
